# Initial kernel scaffold; baseline (speedup 1.0000x reference)
import jax
import jax.numpy as jnp
from jax.experimental import pallas as pl


def kernel(x, edge_index, edge_attr, fc1_W, fc1_b, emb1_1, emb2_1, Wa1, ba1, Wb1, bb1, emb1_2, emb2_2, Wa2, ba2, Wb2, bb2, fc2_W, fc2_b):
    raise NotImplementedError("write your pallas kernel here")


# speedup vs baseline: 8.3734x; 8.3734x over previous
import functools

import jax
import jax.numpy as jnp
from jax import lax
from jax.experimental import pallas as pl
from jax.experimental.pallas import tpu as pltpu
from jax.experimental.pallas import tpu_sc as plsc

N_NODES = 10000
N_PAD = 10240
N_EDGES = 320000
E_PAD = 327680
D = 128
COMBOS = 16
NC, NS = 2, 16
W = 128
EROWS = E_PAD // W
EROWS_PER_WORKER = EROWS // (NC * NS)
ROWS_PER_TILE = N_PAD // NS

_sc_mesh = plsc.VectorSubcoreMesh(core_axis_name="c", subcore_axis_name="s")


@functools.partial(
    pl.kernel,
    out_type=jax.ShapeDtypeStruct((NC, N_PAD, D), jnp.float32),
    mesh=_sc_mesh,
    scratch_types=[
        pltpu.VMEM_SHARED((N_PAD, D), jnp.float32),
        pltpu.VMEM((W,), jnp.int32),
        pltpu.VMEM((W,), jnp.int32),
        pltpu.VMEM((W, D), jnp.float32),
        pltpu.SemaphoreType.DMA,
    ],
)
def _sc_agg(h_hbm, z_hbm, src_hbm, dst_hbm, out_hbm,
            agg_sp, src_v, dst_v, rows_v, sem):
    cid = lax.axis_index("c")
    sid = lax.axis_index("s")
    r0 = sid * ROWS_PER_TILE

    @pl.when(cid == 0)
    def _():
        pltpu.sync_copy(h_hbm.at[pl.ds(r0, ROWS_PER_TILE), :],
                        agg_sp.at[pl.ds(r0, ROWS_PER_TILE), :])

    @pl.when(cid != 0)
    def _():
        pltpu.sync_copy(z_hbm.at[pl.ds(r0, ROWS_PER_TILE), :],
                        agg_sp.at[pl.ds(r0, ROWS_PER_TILE), :])

    e0 = (cid * NS + sid) * EROWS_PER_WORKER * W
    plsc.subcore_barrier()

    def body(i, carry):
        pltpu.sync_copy(src_hbm.at[pl.ds(e0 + i * W, W)], src_v)
        pltpu.sync_copy(dst_hbm.at[pl.ds(e0 + i * W, W)], dst_v)
        pltpu.async_copy(h_hbm.at[src_v], rows_v, sem).wait()
        pltpu.sync_copy(rows_v, agg_sp.at[dst_v], add=True)
        return carry

    lax.fori_loop(0, EROWS_PER_WORKER, body, 0)
    plsc.subcore_barrier()
    pltpu.sync_copy(agg_sp.at[pl.ds(r0, ROWS_PER_TILE), :],
                    out_hbm.at[cid, pl.ds(r0, ROWS_PER_TILE), :])


@functools.partial(
    pl.kernel,
    out_type=jax.ShapeDtypeStruct((NC, N_PAD, D), jnp.float32),
    mesh=_sc_mesh,
    scratch_types=[
        pltpu.VMEM_SHARED((N_PAD, D), jnp.float32),
        pltpu.VMEM_SHARED((2 * COMBOS, D), jnp.float32),
        pltpu.VMEM((W,), jnp.int32),
        pltpu.VMEM((W,), jnp.int32),
        pltpu.VMEM((W, D), jnp.float32),
        pltpu.SemaphoreType.DMA,
    ],
)
def _sc_cnt(oh_hbm, zc_hbm, c_hbm, dst_hbm, out_hbm,
            cnt_sp, oh_sp, c_v, dst_v, rows_v, sem):
    cid = lax.axis_index("c")
    sid = lax.axis_index("s")
    r0 = sid * ROWS_PER_TILE
    pltpu.sync_copy(zc_hbm.at[pl.ds(r0, ROWS_PER_TILE), :],
                    cnt_sp.at[pl.ds(r0, ROWS_PER_TILE), :])

    @pl.when(sid == 0)
    def _():
        pltpu.sync_copy(oh_hbm, oh_sp)

    e0 = (cid * NS + sid) * EROWS_PER_WORKER * W
    plsc.subcore_barrier()

    def body(i, carry):
        pltpu.sync_copy(c_hbm.at[pl.ds(e0 + i * W, W)], c_v)
        pltpu.sync_copy(dst_hbm.at[pl.ds(e0 + i * W, W)], dst_v)
        pltpu.async_copy(oh_sp.at[c_v], rows_v, sem).wait()
        pltpu.sync_copy(rows_v, cnt_sp.at[dst_v], add=True)
        return carry

    lax.fori_loop(0, EROWS_PER_WORKER, body, 0)
    plsc.subcore_barrier()
    pltpu.sync_copy(cnt_sp.at[pl.ds(r0, ROWS_PER_TILE), :],
                    out_hbm.at[cid, pl.ds(r0, ROWS_PER_TILE), :])


BN = 1024


def _fc1_body(x_ref, w_ref, b_ref, o_ref):
    o_ref[:] = jnp.maximum(
        jnp.dot(x_ref[:], w_ref[:], preferred_element_type=jnp.float32)
        + b_ref[:], 0.0)


_fc1 = pl.pallas_call(
    _fc1_body,
    grid=(N_PAD // BN,),
    in_specs=[
        pl.BlockSpec((BN, D), lambda i: (i, 0)),
        pl.BlockSpec((D, D), lambda i: (0, 0)),
        pl.BlockSpec((1, D), lambda i: (0, 0)),
    ],
    out_specs=pl.BlockSpec((BN, D), lambda i: (i, 0)),
    out_shape=jax.ShapeDtypeStruct((N_PAD, D), jnp.float32),
)


def _mlp_body(a_ref, c_ref, t_ref, s_ref, wa_ref, ba_ref, wb_ref, bb_ref,
              *rest, final_relu):
    if final_relu:
        (o_ref,) = rest
    else:
        w2_ref, b2_ref, o_ref = rest
    agg = a_ref[0] + a_ref[1] + s_ref[:]
    cnt = c_ref[0] + c_ref[1]
    agg = agg + jnp.dot(cnt, t_ref[:], preferred_element_type=jnp.float32)
    hid = jnp.maximum(
        jnp.dot(agg, wa_ref[:], preferred_element_type=jnp.float32)
        + ba_ref[:], 0.0)
    y = jnp.dot(hid, wb_ref[:], preferred_element_type=jnp.float32) + bb_ref[:]
    if final_relu:
        y = jnp.maximum(y, 0.0)
    else:
        y = jnp.dot(y, w2_ref[:], preferred_element_type=jnp.float32) + b2_ref[:]
    o_ref[:] = y


def _make_mlp(final_relu):
    in_specs = [
        pl.BlockSpec((2, BN, D), lambda i: (0, i, 0)),
        pl.BlockSpec((2, BN, D), lambda i: (0, i, 0)),
        pl.BlockSpec((D, D), lambda i: (0, 0)),
        pl.BlockSpec((1, D), lambda i: (0, 0)),
        pl.BlockSpec((D, 2 * D), lambda i: (0, 0)),
        pl.BlockSpec((1, 2 * D), lambda i: (0, 0)),
        pl.BlockSpec((2 * D, D), lambda i: (0, 0)),
        pl.BlockSpec((1, D), lambda i: (0, 0)),
    ]
    if not final_relu:
        in_specs += [
            pl.BlockSpec((D, D), lambda i: (0, 0)),
            pl.BlockSpec((1, D), lambda i: (0, 0)),
        ]
    return pl.pallas_call(
        functools.partial(_mlp_body, final_relu=final_relu),
        grid=(N_PAD // BN,),
        in_specs=in_specs,
        out_specs=pl.BlockSpec((BN, D), lambda i: (i, 0)),
        out_shape=jax.ShapeDtypeStruct((N_PAD, D), jnp.float32),
    )


_mlp1 = _make_mlp(final_relu=True)
_mlp2 = _make_mlp(final_relu=False)


def kernel(x, edge_index, edge_attr, fc1_W, fc1_b, emb1_1, emb2_1, Wa1, ba1,
           Wb1, bb1, emb1_2, emb2_2, Wa2, ba2, Wb2, bb2, fc2_W, fc2_b):
    f32 = jnp.float32
    src = edge_index[0]
    dst = edge_index[1]
    combo = edge_attr[:, 0] * 3 + edge_attr[:, 1]

    npad = E_PAD - N_EDGES
    j = jnp.arange(npad, dtype=jnp.int32) % 16
    padrow = N_NODES + j
    src_p = jnp.concatenate([src, padrow])
    dst_p = jnp.concatenate([dst, padrow])
    c_p = jnp.concatenate([combo, COMBOS + j])

    x_p = jnp.pad(x, ((0, N_PAD - N_NODES), (0, 0)))
    zD = jnp.zeros((N_PAD, D), f32)
    onehot = jnp.pad(jnp.eye(COMBOS, dtype=f32),
                     ((0, COMBOS), (0, D - COMBOS)))

    i1 = jnp.repeat(jnp.arange(3), 3)
    i2 = jnp.tile(jnp.arange(3), 3)
    T1 = jnp.concatenate([emb1_1[i1] + emb2_1[i2],
                          jnp.zeros((D - 9, D), f32)], axis=0)
    T2 = jnp.concatenate([emb1_2[i1] + emb2_2[i2],
                          jnp.zeros((D - 9, D), f32)], axis=0)
    s1 = (emb1_1[4] + emb2_1[0])[None, :]
    s2 = (emb1_2[4] + emb2_2[0])[None, :]

    cnt2 = _sc_cnt(onehot, zD, c_p, dst_p)
    h0 = _fc1(x_p, fc1_W, fc1_b[None, :])
    agg1 = _sc_agg(h0, zD, src_p, dst_p)
    h1 = _mlp1(agg1, cnt2, T1, s1, Wa1, ba1[None, :], Wb1, bb1[None, :])
    agg2 = _sc_agg(h1, zD, src_p, dst_p)
    out = _mlp2(agg2, cnt2, T2, s2, Wa2, ba2[None, :], Wb2, bb2[None, :],
                fc2_W, fc2_b[None, :])
    return out[:N_NODES]


# speedup vs baseline: 12.4494x; 1.4868x over previous
import functools

import jax
import jax.numpy as jnp
from jax import lax
from jax.experimental import pallas as pl
from jax.experimental.pallas import tpu as pltpu
from jax.experimental.pallas import tpu_sc as plsc

N_NODES = 10000
N_PAD = 10240
N_EDGES = 320000
E_PAD = 327680
D = 128
COMBOS = 16
NC, NS = 2, 16
W = 128
EROWS = E_PAD // W
EROWS_PER_WORKER = EROWS // (NC * NS)
ROWS_PER_TILE = N_PAD // NS

_sc_mesh = plsc.VectorSubcoreMesh(core_axis_name="c", subcore_axis_name="s")

NWIN = EROWS_PER_WORKER
CHUNK = 20


def _edge_loop(gsrc, src_hbm, dst_hbm, w0, idx_s3, idx_d3,
               buf0, buf1, sem0, sem1, acc_sp):
    def chunk_body(ci, carry):
        pltpu.sync_copy(src_hbm.at[pl.ds(w0 + ci * CHUNK, CHUNK), :, :],
                        idx_s3)
        pltpu.sync_copy(dst_hbm.at[pl.ds(w0 + ci * CHUNK, CHUNK), :, :],
                        idx_d3)
        pltpu.async_copy(gsrc.at[idx_s3.at[0, 0]], buf0, sem0)

        def body(j, c2):
            i0 = 2 * j
            i1 = i0 + 1
            pltpu.make_async_copy(gsrc.at[idx_s3.at[i0, 0]], buf0, sem0).wait()
            pltpu.async_copy(gsrc.at[idx_s3.at[i1, 0]], buf1, sem1)
            pltpu.sync_copy(buf0, acc_sp.at[idx_d3.at[i0, 0]], add=True)
            pltpu.make_async_copy(gsrc.at[idx_s3.at[i1, 0]], buf1, sem1).wait()

            @pl.when(j < CHUNK // 2 - 1)
            def _():
                pltpu.async_copy(gsrc.at[idx_s3.at[i0 + 2, 0]], buf0, sem0)

            pltpu.sync_copy(buf1, acc_sp.at[idx_d3.at[i1, 0]], add=True)
            return c2

        lax.fori_loop(0, CHUNK // 2, body, carry)
        return carry

    lax.fori_loop(0, NWIN // CHUNK, chunk_body, 0)


@functools.partial(
    pl.kernel,
    out_type=jax.ShapeDtypeStruct((NC, N_PAD, D), jnp.float32),
    mesh=_sc_mesh,
    scratch_types=[
        pltpu.VMEM_SHARED((N_PAD, D), jnp.float32),
        pltpu.VMEM((CHUNK, 1, W), jnp.int32),
        pltpu.VMEM((CHUNK, 1, W), jnp.int32),
        pltpu.VMEM((W, D), jnp.float32),
        pltpu.VMEM((W, D), jnp.float32),
        pltpu.SemaphoreType.DMA,
        pltpu.SemaphoreType.DMA,
    ],
)
def _sc_agg(h_hbm, z_hbm, src_hbm, dst_hbm, out_hbm,
            agg_sp, src_v, dst_v, buf0, buf1, sem0, sem1):
    cid = lax.axis_index("c")
    sid = lax.axis_index("s")
    r0 = sid * ROWS_PER_TILE

    @pl.when(cid == 0)
    def _():
        pltpu.sync_copy(h_hbm.at[pl.ds(r0, ROWS_PER_TILE), :],
                        agg_sp.at[pl.ds(r0, ROWS_PER_TILE), :])

    @pl.when(cid != 0)
    def _():
        pltpu.sync_copy(z_hbm.at[pl.ds(r0, ROWS_PER_TILE), :],
                        agg_sp.at[pl.ds(r0, ROWS_PER_TILE), :])

    w0 = (cid * NS + sid) * NWIN
    plsc.subcore_barrier()
    _edge_loop(h_hbm, src_hbm, dst_hbm, w0, src_v, dst_v,
               buf0, buf1, sem0, sem1, agg_sp)
    plsc.subcore_barrier()
    pltpu.sync_copy(agg_sp.at[pl.ds(r0, ROWS_PER_TILE), :],
                    out_hbm.at[cid, pl.ds(r0, ROWS_PER_TILE), :])


@functools.partial(
    pl.kernel,
    out_type=jax.ShapeDtypeStruct((NC, N_PAD, D), jnp.float32),
    mesh=_sc_mesh,
    scratch_types=[
        pltpu.VMEM_SHARED((N_PAD, D), jnp.float32),
        pltpu.VMEM_SHARED((2 * COMBOS, D), jnp.float32),
        pltpu.VMEM((CHUNK, 1, W), jnp.int32),
        pltpu.VMEM((CHUNK, 1, W), jnp.int32),
        pltpu.VMEM((W, D), jnp.float32),
        pltpu.VMEM((W, D), jnp.float32),
        pltpu.SemaphoreType.DMA,
        pltpu.SemaphoreType.DMA,
    ],
)
def _sc_cnt(oh_hbm, zc_hbm, c_hbm, dst_hbm, out_hbm,
            cnt_sp, oh_sp, c_v, dst_v, buf0, buf1, sem0, sem1):
    cid = lax.axis_index("c")
    sid = lax.axis_index("s")
    r0 = sid * ROWS_PER_TILE
    pltpu.sync_copy(zc_hbm.at[pl.ds(r0, ROWS_PER_TILE), :],
                    cnt_sp.at[pl.ds(r0, ROWS_PER_TILE), :])

    @pl.when(sid == 0)
    def _():
        pltpu.sync_copy(oh_hbm, oh_sp)

    w0 = (cid * NS + sid) * NWIN
    plsc.subcore_barrier()
    _edge_loop(oh_sp, c_hbm, dst_hbm, w0, c_v, dst_v,
               buf0, buf1, sem0, sem1, cnt_sp)
    plsc.subcore_barrier()
    pltpu.sync_copy(cnt_sp.at[pl.ds(r0, ROWS_PER_TILE), :],
                    out_hbm.at[cid, pl.ds(r0, ROWS_PER_TILE), :])


BN = 1024


def _fc1_body(x_ref, w_ref, b_ref, o_ref):
    o_ref[:] = jnp.maximum(
        jnp.dot(x_ref[:], w_ref[:], preferred_element_type=jnp.float32)
        + b_ref[:], 0.0)


_fc1 = pl.pallas_call(
    _fc1_body,
    grid=(N_PAD // BN,),
    in_specs=[
        pl.BlockSpec((BN, D), lambda i: (i, 0)),
        pl.BlockSpec((D, D), lambda i: (0, 0)),
        pl.BlockSpec((1, D), lambda i: (0, 0)),
    ],
    out_specs=pl.BlockSpec((BN, D), lambda i: (i, 0)),
    out_shape=jax.ShapeDtypeStruct((N_PAD, D), jnp.float32),
)


def _mlp_body(a_ref, c_ref, t_ref, s_ref, wa_ref, ba_ref, wb_ref, bb_ref,
              *rest, final_relu):
    if final_relu:
        (o_ref,) = rest
    else:
        w2_ref, b2_ref, o_ref = rest
    agg = a_ref[0] + a_ref[1] + s_ref[:]
    cnt = c_ref[0] + c_ref[1]
    agg = agg + jnp.dot(cnt, t_ref[:], preferred_element_type=jnp.float32)
    hid = jnp.maximum(
        jnp.dot(agg, wa_ref[:], preferred_element_type=jnp.float32)
        + ba_ref[:], 0.0)
    y = jnp.dot(hid, wb_ref[:], preferred_element_type=jnp.float32) + bb_ref[:]
    if final_relu:
        y = jnp.maximum(y, 0.0)
    else:
        y = jnp.dot(y, w2_ref[:], preferred_element_type=jnp.float32) + b2_ref[:]
    o_ref[:] = y


def _make_mlp(final_relu):
    in_specs = [
        pl.BlockSpec((2, BN, D), lambda i: (0, i, 0)),
        pl.BlockSpec((2, BN, D), lambda i: (0, i, 0)),
        pl.BlockSpec((D, D), lambda i: (0, 0)),
        pl.BlockSpec((1, D), lambda i: (0, 0)),
        pl.BlockSpec((D, 2 * D), lambda i: (0, 0)),
        pl.BlockSpec((1, 2 * D), lambda i: (0, 0)),
        pl.BlockSpec((2 * D, D), lambda i: (0, 0)),
        pl.BlockSpec((1, D), lambda i: (0, 0)),
    ]
    if not final_relu:
        in_specs += [
            pl.BlockSpec((D, D), lambda i: (0, 0)),
            pl.BlockSpec((1, D), lambda i: (0, 0)),
        ]
    return pl.pallas_call(
        functools.partial(_mlp_body, final_relu=final_relu),
        grid=(N_PAD // BN,),
        in_specs=in_specs,
        out_specs=pl.BlockSpec((BN, D), lambda i: (i, 0)),
        out_shape=jax.ShapeDtypeStruct((N_PAD, D), jnp.float32),
    )


_mlp1 = _make_mlp(final_relu=True)
_mlp2 = _make_mlp(final_relu=False)


def kernel(x, edge_index, edge_attr, fc1_W, fc1_b, emb1_1, emb2_1, Wa1, ba1,
           Wb1, bb1, emb1_2, emb2_2, Wa2, ba2, Wb2, bb2, fc2_W, fc2_b):
    f32 = jnp.float32
    src = edge_index[0]
    dst = edge_index[1]
    combo = edge_attr[:, 0] * 3 + edge_attr[:, 1]

    npad = E_PAD - N_EDGES
    j = jnp.arange(npad, dtype=jnp.int32) % 16
    padrow = N_NODES + j
    src_p = jnp.concatenate([src, padrow]).reshape(EROWS, 1, W)
    dst_p = jnp.concatenate([dst, padrow]).reshape(EROWS, 1, W)
    c_p = jnp.concatenate([combo, COMBOS + j]).reshape(EROWS, 1, W)

    x_p = jnp.pad(x, ((0, N_PAD - N_NODES), (0, 0)))
    zD = jnp.zeros((N_PAD, D), f32)
    onehot = jnp.pad(jnp.eye(COMBOS, dtype=f32),
                     ((0, COMBOS), (0, D - COMBOS)))

    i1 = jnp.repeat(jnp.arange(3), 3)
    i2 = jnp.tile(jnp.arange(3), 3)
    T1 = jnp.concatenate([emb1_1[i1] + emb2_1[i2],
                          jnp.zeros((D - 9, D), f32)], axis=0)
    T2 = jnp.concatenate([emb1_2[i1] + emb2_2[i2],
                          jnp.zeros((D - 9, D), f32)], axis=0)
    s1 = (emb1_1[4] + emb2_1[0])[None, :]
    s2 = (emb1_2[4] + emb2_2[0])[None, :]

    cnt2 = _sc_cnt(onehot, zD, c_p, dst_p)
    h0 = _fc1(x_p, fc1_W, fc1_b[None, :])
    agg1 = _sc_agg(h0, zD, src_p, dst_p)
    h1 = _mlp1(agg1, cnt2, T1, s1, Wa1, ba1[None, :], Wb1, bb1[None, :])
    agg2 = _sc_agg(h1, zD, src_p, dst_p)
    out = _mlp2(agg2, cnt2, T2, s2, Wa2, ba2[None, :], Wb2, bb2[None, :],
                fc2_W, fc2_b[None, :])
    return out[:N_NODES]


# speedup vs baseline: 16.4600x; 1.3222x over previous
import functools

import jax
import jax.numpy as jnp
from jax import lax
from jax.experimental import pallas as pl
from jax.experimental.pallas import tpu as pltpu
from jax.experimental.pallas import tpu_sc as plsc

N_NODES = 10000
N_PAD = 10240
N_EDGES = 320000
E_PAD = 327680
D = 128
COMBOS = 16
NC, NS = 2, 16
W = 128
EROWS = E_PAD // W
EROWS_PER_WORKER = EROWS // (NC * NS)
ROWS_PER_TILE = N_PAD // NS

_sc_mesh = plsc.VectorSubcoreMesh(core_axis_name="c", subcore_axis_name="s")

NWIN = EROWS_PER_WORKER
CHUNK = 20


def _edge_loop(gsrc, src_hbm, dst_hbm, w0, idx_s3, idx_d3,
               buf0, buf1, sem0, sem1, acc_sp):
    def chunk_body(ci, carry):
        pltpu.sync_copy(src_hbm.at[pl.ds(w0 + ci * CHUNK, CHUNK), :, :],
                        idx_s3)
        pltpu.sync_copy(dst_hbm.at[pl.ds(w0 + ci * CHUNK, CHUNK), :, :],
                        idx_d3)
        pltpu.async_copy(gsrc.at[idx_s3.at[0, 0]], buf0, sem0)

        def body(j, c2):
            i0 = 2 * j
            i1 = i0 + 1
            pltpu.make_async_copy(gsrc.at[idx_s3.at[i0, 0]], buf0, sem0).wait()
            pltpu.async_copy(gsrc.at[idx_s3.at[i1, 0]], buf1, sem1)
            pltpu.sync_copy(buf0, acc_sp.at[idx_d3.at[i0, 0]], add=True)
            pltpu.make_async_copy(gsrc.at[idx_s3.at[i1, 0]], buf1, sem1).wait()

            @pl.when(j < CHUNK // 2 - 1)
            def _():
                pltpu.async_copy(gsrc.at[idx_s3.at[i0 + 2, 0]], buf0, sem0)

            pltpu.sync_copy(buf1, acc_sp.at[idx_d3.at[i1, 0]], add=True)
            return c2

        lax.fori_loop(0, CHUNK // 2, body, carry)
        return carry

    lax.fori_loop(0, NWIN // CHUNK, chunk_body, 0)


@functools.partial(
    pl.kernel,
    out_type=jax.ShapeDtypeStruct((NC, N_PAD, D), jnp.float32),
    mesh=_sc_mesh,
    scratch_types=[
        pltpu.VMEM_SHARED((N_PAD, D), jnp.float32),
        pltpu.VMEM((CHUNK, 1, W), jnp.int32),
        pltpu.VMEM((CHUNK, 1, W), jnp.int32),
        pltpu.VMEM((W, D), jnp.float32),
        pltpu.VMEM((W, D), jnp.float32),
        pltpu.SemaphoreType.DMA,
        pltpu.SemaphoreType.DMA,
    ],
)
def _sc_agg(h_hbm, z_hbm, src_hbm, dst_hbm, out_hbm,
            agg_sp, src_v, dst_v, buf0, buf1, sem0, sem1):
    cid = lax.axis_index("c")
    sid = lax.axis_index("s")
    r0 = sid * ROWS_PER_TILE

    @pl.when(cid == 0)
    def _():
        pltpu.sync_copy(h_hbm.at[pl.ds(r0, ROWS_PER_TILE), :],
                        agg_sp.at[pl.ds(r0, ROWS_PER_TILE), :])

    @pl.when(cid != 0)
    def _():
        pltpu.sync_copy(z_hbm.at[pl.ds(r0, ROWS_PER_TILE), :],
                        agg_sp.at[pl.ds(r0, ROWS_PER_TILE), :])

    w0 = (cid * NS + sid) * NWIN
    plsc.subcore_barrier()
    _edge_loop(h_hbm, src_hbm, dst_hbm, w0, src_v, dst_v,
               buf0, buf1, sem0, sem1, agg_sp)
    plsc.subcore_barrier()
    pltpu.sync_copy(agg_sp.at[pl.ds(r0, ROWS_PER_TILE), :],
                    out_hbm.at[cid, pl.ds(r0, ROWS_PER_TILE), :])


@functools.partial(
    pl.kernel,
    out_type=jax.ShapeDtypeStruct((NC, N_PAD // 8, D), jnp.float32),
    mesh=_sc_mesh,
    scratch_types=[
        pltpu.VMEM_SHARED((N_PAD, COMBOS), jnp.float32),
        pltpu.VMEM_SHARED((2 * COMBOS, COMBOS), jnp.float32),
        pltpu.VMEM((CHUNK, 1, W), jnp.int32),
        pltpu.VMEM((CHUNK, 1, W), jnp.int32),
        pltpu.VMEM((W, COMBOS), jnp.float32),
        pltpu.VMEM((W, COMBOS), jnp.float32),
        pltpu.VMEM((2 * COMBOS, COMBOS), jnp.float32),
        pltpu.VMEM((W, COMBOS), jnp.float32),
        pltpu.VMEM((COMBOS, D), jnp.float32),
        pltpu.SemaphoreType.DMA,
        pltpu.SemaphoreType.DMA,
    ],
)
def _sc_cnt(c_hbm, dst_hbm, out_hbm, cnt_sp, oh_sp, c_v, dst_v,
            buf0, buf1, idbuf, zbuf, obuf, sem0, sem1):
    cid = lax.axis_index("c")
    sid = lax.axis_index("s")
    r0 = sid * ROWS_PER_TILE
    lane = lax.iota(jnp.int32, 16)
    for r in range(COMBOS):
        idbuf[r, :] = jnp.where(lane == r, 1.0, 0.0).astype(jnp.float32)
    zrow = jnp.zeros((COMBOS,), jnp.float32)
    for r in range(COMBOS, 2 * COMBOS):
        idbuf[r, :] = zrow

    @pl.when(sid == 0)
    def _():
        pltpu.sync_copy(idbuf, oh_sp)

    def zb(i, carry):
        zbuf[i, :] = zrow
        return carry

    lax.fori_loop(0, W, zb, 0)
    for z in range(ROWS_PER_TILE // W):
        pltpu.sync_copy(zbuf, cnt_sp.at[pl.ds(r0 + z * W, W), :])

    w0 = (cid * NS + sid) * NWIN
    plsc.subcore_barrier()
    _edge_loop(oh_sp, c_hbm, dst_hbm, w0, c_v, dst_v,
               buf0, buf1, sem0, sem1, cnt_sp)
    plsc.subcore_barrier()
    for z in range(ROWS_PER_TILE // W):
        pltpu.sync_copy(cnt_sp.at[pl.ds(r0 + z * W, W), :], zbuf)
        for r in range(W):
            obuf[r // 8, pl.ds((r % 8) * COMBOS, COMBOS)] = zbuf[r, :]
        orow = pl.multiple_of((r0 + z * W) // 8, 8)
        pltpu.sync_copy(obuf, out_hbm.at[cid, pl.ds(orow, COMBOS), :])


BN = 1024


def _fc1_body(x_ref, w_ref, b_ref, o_ref):
    o_ref[:] = jnp.maximum(
        jnp.dot(x_ref[:], w_ref[:], preferred_element_type=jnp.float32)
        + b_ref[:], 0.0)


_fc1 = pl.pallas_call(
    _fc1_body,
    grid=(N_PAD // BN,),
    in_specs=[
        pl.BlockSpec((BN, D), lambda i: (i, 0)),
        pl.BlockSpec((D, D), lambda i: (0, 0)),
        pl.BlockSpec((1, D), lambda i: (0, 0)),
    ],
    out_specs=pl.BlockSpec((BN, D), lambda i: (i, 0)),
    out_shape=jax.ShapeDtypeStruct((N_PAD, D), jnp.float32),
)


def _mlp_body(a_ref, c_ref, t_ref, s_ref, wa_ref, ba_ref, wb_ref, bb_ref,
              *rest, final_relu):
    if final_relu:
        (o_ref,) = rest
    else:
        w2_ref, b2_ref, o_ref = rest
    agg = a_ref[0] + a_ref[1] + s_ref[:]
    cnt = c_ref[0] + c_ref[1]
    agg = agg + jnp.dot(cnt, t_ref[:], preferred_element_type=jnp.float32)
    hid = jnp.maximum(
        jnp.dot(agg, wa_ref[:], preferred_element_type=jnp.float32)
        + ba_ref[:], 0.0)
    y = jnp.dot(hid, wb_ref[:], preferred_element_type=jnp.float32) + bb_ref[:]
    if final_relu:
        y = jnp.maximum(y, 0.0)
    else:
        y = jnp.dot(y, w2_ref[:], preferred_element_type=jnp.float32) + b2_ref[:]
    o_ref[:] = y


def _make_mlp(final_relu):
    in_specs = [
        pl.BlockSpec((2, BN, D), lambda i: (0, i, 0)),
        pl.BlockSpec((2, BN, COMBOS), lambda i: (0, i, 0)),
        pl.BlockSpec((COMBOS, D), lambda i: (0, 0)),
        pl.BlockSpec((1, D), lambda i: (0, 0)),
        pl.BlockSpec((D, 2 * D), lambda i: (0, 0)),
        pl.BlockSpec((1, 2 * D), lambda i: (0, 0)),
        pl.BlockSpec((2 * D, D), lambda i: (0, 0)),
        pl.BlockSpec((1, D), lambda i: (0, 0)),
    ]
    if not final_relu:
        in_specs += [
            pl.BlockSpec((D, D), lambda i: (0, 0)),
            pl.BlockSpec((1, D), lambda i: (0, 0)),
        ]
    return pl.pallas_call(
        functools.partial(_mlp_body, final_relu=final_relu),
        grid=(N_PAD // BN,),
        in_specs=in_specs,
        out_specs=pl.BlockSpec((BN, D), lambda i: (i, 0)),
        out_shape=jax.ShapeDtypeStruct((N_PAD, D), jnp.float32),
    )


_mlp1 = _make_mlp(final_relu=True)
_mlp2 = _make_mlp(final_relu=False)


def kernel(x, edge_index, edge_attr, fc1_W, fc1_b, emb1_1, emb2_1, Wa1, ba1,
           Wb1, bb1, emb1_2, emb2_2, Wa2, ba2, Wb2, bb2, fc2_W, fc2_b):
    f32 = jnp.float32
    src = edge_index[0]
    dst = edge_index[1]
    combo = edge_attr[:, 0] * 3 + edge_attr[:, 1]

    npad = E_PAD - N_EDGES
    j = jnp.arange(npad, dtype=jnp.int32) % 16
    padrow = N_NODES + j
    src_p = jnp.concatenate([src, padrow]).reshape(EROWS, 1, W)
    dst_p = jnp.concatenate([dst, padrow]).reshape(EROWS, 1, W)
    c_p = jnp.concatenate([combo, COMBOS + j]).reshape(EROWS, 1, W)

    x_p = jnp.pad(x, ((0, N_PAD - N_NODES), (0, 0)))
    zD = jnp.zeros((N_PAD, D), f32)

    i1 = jnp.repeat(jnp.arange(3), 3)
    i2 = jnp.tile(jnp.arange(3), 3)
    T1 = jnp.concatenate([emb1_1[i1] + emb2_1[i2],
                          jnp.zeros((COMBOS - 9, D), f32)], axis=0)
    T2 = jnp.concatenate([emb1_2[i1] + emb2_2[i2],
                          jnp.zeros((COMBOS - 9, D), f32)], axis=0)
    s1 = (emb1_1[4] + emb2_1[0])[None, :]
    s2 = (emb1_2[4] + emb2_2[0])[None, :]

    cnt2 = _sc_cnt(c_p, dst_p).reshape(NC, N_PAD, COMBOS)
    x_p = x_p + cnt2[0, 0, 0] * 0.0
    h0 = _fc1(x_p, fc1_W, fc1_b[None, :])
    agg1 = _sc_agg(h0, zD, src_p, dst_p)
    h1 = _mlp1(agg1, cnt2, T1, s1, Wa1, ba1[None, :], Wb1, bb1[None, :])
    agg2 = _sc_agg(h1, zD, src_p, dst_p)
    out = _mlp2(agg2, cnt2, T2, s2, Wa2, ba2[None, :], Wb2, bb2[None, :],
                fc2_W, fc2_b[None, :])
    return out[:N_NODES]


# speedup vs baseline: 17.7051x; 1.0756x over previous
import functools

import jax
import jax.numpy as jnp
from jax import lax
from jax.experimental import pallas as pl
from jax.experimental.pallas import tpu as pltpu
from jax.experimental.pallas import tpu_sc as plsc

N_NODES = 10000
N_PAD = 10240
N_EDGES = 320000
E_PAD = 327680
D = 128
COMBOS = 16
NC, NS = 2, 16
W = 128
EROWS = E_PAD // W
EROWS_PER_WORKER = EROWS // (NC * NS)
ROWS_PER_TILE = N_PAD // NS

_sc_mesh = plsc.VectorSubcoreMesh(core_axis_name="c", subcore_axis_name="s")

NWIN = EROWS_PER_WORKER
CHUNK = 40


def _edge_loop(gsrc, src_hbm, dst_hbm, w0, idx_s3, idx_d3,
               buf0, buf1, sem0, sem1, acc_sp):
    def chunk_body(ci, carry):
        pltpu.sync_copy(src_hbm.at[pl.ds(w0 + ci * CHUNK, CHUNK), :, :],
                        idx_s3)
        pltpu.sync_copy(dst_hbm.at[pl.ds(w0 + ci * CHUNK, CHUNK), :, :],
                        idx_d3)
        pltpu.async_copy(gsrc.at[idx_s3.at[0, 0]], buf0, sem0)

        def body(j, c2):
            i0 = 2 * j
            i1 = i0 + 1
            pltpu.make_async_copy(gsrc.at[idx_s3.at[i0, 0]], buf0, sem0).wait()
            pltpu.async_copy(gsrc.at[idx_s3.at[i1, 0]], buf1, sem1)
            pltpu.sync_copy(buf0, acc_sp.at[idx_d3.at[i0, 0]], add=True)
            pltpu.make_async_copy(gsrc.at[idx_s3.at[i1, 0]], buf1, sem1).wait()

            @pl.when(j < CHUNK // 2 - 1)
            def _():
                pltpu.async_copy(gsrc.at[idx_s3.at[i0 + 2, 0]], buf0, sem0)

            pltpu.sync_copy(buf1, acc_sp.at[idx_d3.at[i1, 0]], add=True)
            return c2

        lax.fori_loop(0, CHUNK // 2, body, carry)
        return carry

    lax.fori_loop(0, NWIN // CHUNK, chunk_body, 0)


@functools.partial(
    pl.kernel,
    out_type=jax.ShapeDtypeStruct((NC, N_PAD, D), jnp.float32),
    mesh=_sc_mesh,
    scratch_types=[
        pltpu.VMEM_SHARED((N_PAD, D), jnp.float32),
        pltpu.VMEM((CHUNK, 1, W), jnp.int32),
        pltpu.VMEM((CHUNK, 1, W), jnp.int32),
        pltpu.VMEM((W, D), jnp.float32),
        pltpu.VMEM((W, D), jnp.float32),
        pltpu.SemaphoreType.DMA,
        pltpu.SemaphoreType.DMA,
    ],
)
def _sc_agg(h_hbm, z_hbm, src_hbm, dst_hbm, out_hbm,
            agg_sp, src_v, dst_v, buf0, buf1, sem0, sem1):
    cid = lax.axis_index("c")
    sid = lax.axis_index("s")
    r0 = sid * ROWS_PER_TILE

    @pl.when(cid == 0)
    def _():
        pltpu.sync_copy(h_hbm.at[pl.ds(r0, ROWS_PER_TILE), :],
                        agg_sp.at[pl.ds(r0, ROWS_PER_TILE), :])

    @pl.when(cid != 0)
    def _():
        pltpu.sync_copy(z_hbm.at[pl.ds(r0, ROWS_PER_TILE), :],
                        agg_sp.at[pl.ds(r0, ROWS_PER_TILE), :])

    w0 = (cid * NS + sid) * NWIN
    plsc.subcore_barrier()
    _edge_loop(h_hbm, src_hbm, dst_hbm, w0, src_v, dst_v,
               buf0, buf1, sem0, sem1, agg_sp)
    plsc.subcore_barrier()
    pltpu.sync_copy(agg_sp.at[pl.ds(r0, ROWS_PER_TILE), :],
                    out_hbm.at[cid, pl.ds(r0, ROWS_PER_TILE), :])


@functools.partial(
    pl.kernel,
    out_type=jax.ShapeDtypeStruct((NC, N_PAD // 8, D), jnp.float32),
    mesh=_sc_mesh,
    scratch_types=[
        pltpu.VMEM_SHARED((N_PAD, COMBOS), jnp.float32),
        pltpu.VMEM_SHARED((2 * COMBOS, COMBOS), jnp.float32),
        pltpu.VMEM((CHUNK, 1, W), jnp.int32),
        pltpu.VMEM((CHUNK, 1, W), jnp.int32),
        pltpu.VMEM((W, COMBOS), jnp.float32),
        pltpu.VMEM((W, COMBOS), jnp.float32),
        pltpu.VMEM((2 * COMBOS, COMBOS), jnp.float32),
        pltpu.VMEM((W, COMBOS), jnp.float32),
        pltpu.VMEM((COMBOS, D), jnp.float32),
        pltpu.SemaphoreType.DMA,
        pltpu.SemaphoreType.DMA,
    ],
)
def _sc_cnt(c_hbm, dst_hbm, out_hbm, cnt_sp, oh_sp, c_v, dst_v,
            buf0, buf1, idbuf, zbuf, obuf, sem0, sem1):
    cid = lax.axis_index("c")
    sid = lax.axis_index("s")
    r0 = sid * ROWS_PER_TILE
    lane = lax.iota(jnp.int32, 16)
    for r in range(COMBOS):
        idbuf[r, :] = jnp.where(lane == r, 1.0, 0.0).astype(jnp.float32)
    zrow = jnp.zeros((COMBOS,), jnp.float32)
    for r in range(COMBOS, 2 * COMBOS):
        idbuf[r, :] = zrow

    @pl.when(sid == 0)
    def _():
        pltpu.sync_copy(idbuf, oh_sp)

    def zb(i, carry):
        zbuf[i, :] = zrow
        return carry

    lax.fori_loop(0, W, zb, 0)
    for z in range(ROWS_PER_TILE // W):
        pltpu.sync_copy(zbuf, cnt_sp.at[pl.ds(r0 + z * W, W), :])

    w0 = (cid * NS + sid) * NWIN
    plsc.subcore_barrier()
    _edge_loop(oh_sp, c_hbm, dst_hbm, w0, c_v, dst_v,
               buf0, buf1, sem0, sem1, cnt_sp)
    plsc.subcore_barrier()
    for z in range(ROWS_PER_TILE // W):
        pltpu.sync_copy(cnt_sp.at[pl.ds(r0 + z * W, W), :], zbuf)
        for r in range(W):
            obuf[r // 8, pl.ds((r % 8) * COMBOS, COMBOS)] = zbuf[r, :]
        orow = pl.multiple_of((r0 + z * W) // 8, 8)
        pltpu.sync_copy(obuf, out_hbm.at[cid, pl.ds(orow, COMBOS), :])


BN = 2048


def _fc1_body(x_ref, w_ref, b_ref, o_ref):
    o_ref[:] = jnp.maximum(
        jnp.dot(x_ref[:], w_ref[:], preferred_element_type=jnp.float32)
        + b_ref[:], 0.0)


_fc1 = pl.pallas_call(
    _fc1_body,
    grid=(N_PAD // BN,),
    in_specs=[
        pl.BlockSpec((BN, D), lambda i: (i, 0)),
        pl.BlockSpec((D, D), lambda i: (0, 0)),
        pl.BlockSpec((1, D), lambda i: (0, 0)),
    ],
    out_specs=pl.BlockSpec((BN, D), lambda i: (i, 0)),
    out_shape=jax.ShapeDtypeStruct((N_PAD, D), jnp.float32),
)


def _mlp_body(a_ref, c_ref, t_ref, s_ref, wa_ref, ba_ref, wb_ref, bb_ref,
              *rest, final_relu):
    if final_relu:
        (o_ref,) = rest
    else:
        w2_ref, b2_ref, o_ref = rest
    agg = a_ref[0] + a_ref[1] + s_ref[:]
    cnt = c_ref[0] + c_ref[1]
    agg = agg + jnp.dot(cnt, t_ref[:], preferred_element_type=jnp.float32)
    hid = jnp.maximum(
        jnp.dot(agg, wa_ref[:], preferred_element_type=jnp.float32)
        + ba_ref[:], 0.0)
    y = jnp.dot(hid, wb_ref[:], preferred_element_type=jnp.float32) + bb_ref[:]
    if final_relu:
        y = jnp.maximum(y, 0.0)
    else:
        y = jnp.dot(y, w2_ref[:], preferred_element_type=jnp.float32) + b2_ref[:]
    o_ref[:] = y


def _make_mlp(final_relu):
    in_specs = [
        pl.BlockSpec((2, BN, D), lambda i: (0, i, 0)),
        pl.BlockSpec((2, BN, COMBOS), lambda i: (0, i, 0)),
        pl.BlockSpec((COMBOS, D), lambda i: (0, 0)),
        pl.BlockSpec((1, D), lambda i: (0, 0)),
        pl.BlockSpec((D, 2 * D), lambda i: (0, 0)),
        pl.BlockSpec((1, 2 * D), lambda i: (0, 0)),
        pl.BlockSpec((2 * D, D), lambda i: (0, 0)),
        pl.BlockSpec((1, D), lambda i: (0, 0)),
    ]
    if not final_relu:
        in_specs += [
            pl.BlockSpec((D, D), lambda i: (0, 0)),
            pl.BlockSpec((1, D), lambda i: (0, 0)),
        ]
    out_rows = N_PAD if final_relu else N_NODES
    return pl.pallas_call(
        functools.partial(_mlp_body, final_relu=final_relu),
        grid=(N_PAD // BN,),
        in_specs=in_specs,
        out_specs=pl.BlockSpec((BN, D), lambda i: (i, 0)),
        out_shape=jax.ShapeDtypeStruct((out_rows, D), jnp.float32),
    )


_mlp1 = _make_mlp(final_relu=True)
_mlp2 = _make_mlp(final_relu=False)


def kernel(x, edge_index, edge_attr, fc1_W, fc1_b, emb1_1, emb2_1, Wa1, ba1,
           Wb1, bb1, emb1_2, emb2_2, Wa2, ba2, Wb2, bb2, fc2_W, fc2_b):
    f32 = jnp.float32
    src = edge_index[0]
    dst = edge_index[1]
    combo = edge_attr[:, 0] * 3 + edge_attr[:, 1]

    npad = E_PAD - N_EDGES
    j = jnp.arange(npad, dtype=jnp.int32) % 16
    padrow = N_NODES + j
    src_p = jnp.concatenate([src, padrow]).reshape(EROWS, 1, W)
    dst_p = jnp.concatenate([dst, padrow]).reshape(EROWS, 1, W)
    c_p = jnp.concatenate([combo, COMBOS + j]).reshape(EROWS, 1, W)

    x_p = jnp.pad(x, ((0, N_PAD - N_NODES), (0, 0)))
    zD = jnp.zeros((N_PAD, D), f32)

    i1 = jnp.repeat(jnp.arange(3), 3)
    i2 = jnp.tile(jnp.arange(3), 3)
    T1 = jnp.concatenate([emb1_1[i1] + emb2_1[i2],
                          jnp.zeros((COMBOS - 9, D), f32)], axis=0)
    T2 = jnp.concatenate([emb1_2[i1] + emb2_2[i2],
                          jnp.zeros((COMBOS - 9, D), f32)], axis=0)
    s1 = (emb1_1[4] + emb2_1[0])[None, :]
    s2 = (emb1_2[4] + emb2_2[0])[None, :]

    cnt2 = _sc_cnt(c_p, dst_p).reshape(NC, N_PAD, COMBOS)
    h0 = _fc1(x_p, fc1_W, fc1_b[None, :])
    zD1 = zD.at[0, 0].set(cnt2[0, 0, 0] * 0.0)
    agg1 = _sc_agg(h0, zD1, src_p, dst_p)
    h1 = _mlp1(agg1, cnt2, T1, s1, Wa1, ba1[None, :], Wb1, bb1[None, :])
    agg2 = _sc_agg(h1, zD, src_p, dst_p)
    return _mlp2(agg2, cnt2, T2, s2, Wa2, ba2[None, :], Wb2, bb2[None, :],
                 fc2_W, fc2_b[None, :])
